# Initial kernel scaffold; baseline (speedup 1.0000x reference)
import jax
import jax.numpy as jnp
from jax.experimental import pallas as pl


def kernel(user_id, item_ids, edge_index, users_emb, items_emb):
    raise NotImplementedError("write your pallas kernel here")


# speedup vs baseline: 9.6352x; 9.6352x over previous
import jax
import jax.numpy as jnp
from jax import lax
from jax.experimental import pallas as pl
from jax.experimental.pallas import tpu as pltpu
from jax.experimental.pallas import tpu_sc as plsc

NUM_USERS = 10000
NUM_ITEMS = 40000
N_NODES = NUM_USERS + NUM_ITEMS
H = 32
K_LAYERS = 3
BATCH = 4096

NC = 2
NS = 16
NP = 50176
DUMP = N_NODES
NPT = NP // NS
DCH = 112
NDCH = NPT // DCH

E = 800000
E_PAD = 819200
EPT = E_PAD // NS
B = 128
JB = 16
CH = JB * B
NCH = EPT // CH
RPT = EPT // B
BPT = BATCH // NS


def _sc_body(row2d, col1d, e0, uid, iid, out_u, out_i, t_hbm, p_hbm,
             acc_sp, d2_sp, colv, ocolv, rowv, rbuf, obuf, zv, av, tv,
             d2c, pbc, uidxv, iidxv, puv, piv):
    c = lax.axis_index("c")
    s = lax.axis_index("s")
    off = c * NP
    nbase0 = s * NPT

    zeros16 = jnp.zeros((16,), jnp.float32)
    ones16 = jnp.ones((16,), jnp.float32)
    lane = lax.iota(jnp.int32, 16)
    lane0 = lane == 0

    def _init_bufs(i, _):
        zv[i % DCH, pl.ds(0, 16)] = zeros16
        zv[i % DCH, pl.ds(16, 16)] = zeros16
        rbuf[i, pl.ds(0, 16)] = ones16
        rbuf[i, pl.ds(16, 16)] = ones16
        return 0
    lax.fori_loop(0, B, _init_bufs, 0)

    def _zero_ch(ch, _):
        pltpu.sync_copy(zv, acc_sp.at[pl.ds(nbase0 + ch * DCH, DCH)])
        return 0
    lax.fori_loop(0, NDCH, _zero_ch, 0)
    plsc.subcore_barrier()

    def _deg_chunk(m, _):
        pltpu.sync_copy(row2d.at[pl.ds(s * RPT + m * JB, JB)], rowv)

        def _deg_j(j, _):
            pltpu.sync_copy(rbuf, acc_sp.at[rowv.at[j]], add=True)
            return 0
        lax.fori_loop(0, JB, _deg_j, 0)
        return 0
    lax.fori_loop(0, NCH, _deg_chunk, 0)
    plsc.subcore_barrier()

    def _c_chunk(ch, _):
        nb = nbase0 + ch * DCH
        pltpu.sync_copy(acc_sp.at[pl.ds(nb, DCH)], av)
        pltpu.sync_copy(zv, acc_sp.at[pl.ds(nb, DCH)])
        pltpu.sync_copy(e0.at[pl.ds(off + nb, DCH)], obuf.at[pl.ds(0, DCH)])

        def _row_c(n, _):
            d = av[n, pl.ds(0, 16)]
            gb = jnp.full((16,), nb + n, jnp.int32)
            real = (d > 0.5) & (gb < N_NODES)
            xi = lax.bitcast_convert_type(d, jnp.int32)
            y = lax.bitcast_convert_type(0x5F3759DF - (xi >> 1), jnp.float32)
            hx = 0.5 * d
            y = y * (1.5 - hx * y * y)
            y = y * (1.5 - hx * y * y)
            y = y * (1.5 - hx * y * y)
            d1 = jnp.where(real, y, 0.0)
            tv[n, pl.ds(0, 16)] = obuf[n, pl.ds(0, 16)] * d1
            tv[n, pl.ds(16, 16)] = obuf[n, pl.ds(16, 16)] * d1
            nloc = jnp.full((16,), n, jnp.int32)
            plsc.store_scatter(d2c, [nloc], d1 * d1, mask=lane0)
            plsc.store_scatter(pbc, [nloc], d * d1, mask=lane0)
            return 0
        lax.fori_loop(0, DCH, _row_c, 0)
        pltpu.sync_copy(tv, t_hbm.at[pl.ds(off + nb, DCH)])
        pltpu.sync_copy(d2c, d2_sp.at[pl.ds(nb, DCH)])
        pltpu.sync_copy(pbc, p_hbm.at[pl.ds(off + nb, DCH)])
        return 0
    lax.fori_loop(0, NDCH, _c_chunk, 0)

    def _dense_pass():
        def _d_chunk(ch, _):
            nb = nbase0 + ch * DCH
            pltpu.sync_copy(acc_sp.at[pl.ds(nb, DCH)], av)
            pltpu.sync_copy(zv, acc_sp.at[pl.ds(nb, DCH)])
            pltpu.sync_copy(d2_sp.at[pl.ds(nb, DCH)], d2c)

            def _scale_grp(g, _):
                sv = d2c[pl.ds(g * 16, 16)]
                for l in range(16):
                    n = g * 16 + l
                    sc = sv[l]
                    tv[n, pl.ds(0, 16)] = av[n, pl.ds(0, 16)] * sc
                    tv[n, pl.ds(16, 16)] = av[n, pl.ds(16, 16)] * sc
                return 0
            lax.fori_loop(0, DCH // 16, _scale_grp, 0)
            pltpu.sync_copy(tv, t_hbm.at[pl.ds(off + nb, DCH)])
            return 0
        lax.fori_loop(0, NDCH, _d_chunk, 0)

    plsc.subcore_barrier()

    pltpu.sync_copy(uid.at[pl.ds(s * BPT, BPT)], uidxv)
    pltpu.sync_copy(iid.at[pl.ds(s * BPT, BPT)], iidxv)

    def _off_idx(q, _):
        uidxv[pl.ds(q * 16, 16)] = uidxv[pl.ds(q * 16, 16)] + off
        iidxv[pl.ds(q * 16, 16)] = iidxv[pl.ds(q * 16, 16)] + off
        return 0
    lax.fori_loop(0, BPT // 16, _off_idx, 0)

    for q in range(BPT // B):
        pltpu.sync_copy(p_hbm.at[uidxv.at[pl.ds(q * B, B)]],
                        puv.at[pl.ds(q * B, B)])
        pltpu.sync_copy(p_hbm.at[iidxv.at[pl.ds(q * B, B)]],
                        piv.at[pl.ds(q * B, B)])
        pltpu.sync_copy(e0.at[uidxv.at[pl.ds(q * B, B)]], rbuf)
        pltpu.sync_copy(rbuf, out_u.at[c, pl.ds(s * BPT + q * B, B)])
        pltpu.sync_copy(e0.at[iidxv.at[pl.ds(q * B, B)]], rbuf)
        pltpu.sync_copy(rbuf, out_i.at[c, pl.ds(s * BPT + q * B, B)])

    def _layer(k, _):
        fac = jnp.where(k == K_LAYERS - 1, jnp.float32(0.25), jnp.float32(1.0))

        def _edge_step(jj, _):
            m = jj // JB
            j = jj - m * JB

            @pl.when(j == 0)
            def _load_chunk():
                pltpu.sync_copy(row2d.at[pl.ds(s * RPT + m * JB, JB)], rowv)
                pltpu.sync_copy(col1d.at[pl.ds(s * EPT + m * CH, CH)], colv)

                def _off_col(q, _):
                    ocolv[pl.ds(q * 16, 16)] = colv[pl.ds(q * 16, 16)] + off
                    return 0
                lax.fori_loop(0, CH // 16, _off_col, 0)

            pltpu.sync_copy(t_hbm.at[ocolv.at[pl.ds(j * B, B)]], rbuf)
            pltpu.sync_copy(rbuf, acc_sp.at[rowv.at[j]], add=True)
            return 0
        lax.fori_loop(0, RPT, _edge_step, 0)
        plsc.subcore_barrier()

        _dense_pass()
        plsc.subcore_barrier()

        for q in range(BPT // B):
            for which in range(2):
                idxv = uidxv if which == 0 else iidxv
                pv_ref = puv if which == 0 else piv
                out_ref = out_u if which == 0 else out_i
                pltpu.sync_copy(t_hbm.at[idxv.at[pl.ds(q * B, B)]], rbuf)
                pltpu.sync_copy(out_ref.at[c, pl.ds(s * BPT + q * B, B)],
                                obuf.at[pl.ds(0, B)])

                def _acc_out(g, _):
                    pv = pv_ref[pl.ds(q * B + g * 16, 16)]
                    for l in range(16):
                        r = g * 16 + l
                        pr = pv[l]
                        for hh in (0, 16):
                            x = obuf[r, pl.ds(hh, 16)] + pr * rbuf[r, pl.ds(hh, 16)]
                            obuf[r, pl.ds(hh, 16)] = x * fac
                    return 0
                lax.fori_loop(0, B // 16, _acc_out, 0)
                pltpu.sync_copy(obuf.at[pl.ds(0, B)],
                                out_ref.at[c, pl.ds(s * BPT + q * B, B)])
        return 0
    lax.fori_loop(0, K_LAYERS, _layer, 0)


_sc_call = pl.kernel(
    _sc_body,
    out_type=[
        jax.ShapeDtypeStruct((NC, BATCH, H), jnp.float32),
        jax.ShapeDtypeStruct((NC, BATCH, H), jnp.float32),
        jax.ShapeDtypeStruct((NC * NP, H), jnp.float32),
        jax.ShapeDtypeStruct((NC * NP,), jnp.float32),
    ],
    mesh=plsc.VectorSubcoreMesh(core_axis_name="c", subcore_axis_name="s",
                                num_cores=NC, num_subcores=NS),
    compiler_params=pltpu.CompilerParams(use_tc_tiling_on_sc=False,
                                         needs_layout_passes=False),
    scratch_types=[
        pltpu.VMEM_SHARED((NP, H), jnp.float32),
        pltpu.VMEM_SHARED((NP,), jnp.float32),
        pltpu.VMEM((CH,), jnp.int32),
        pltpu.VMEM((CH,), jnp.int32),
        pltpu.VMEM((JB, B), jnp.int32),
        pltpu.VMEM((B, H), jnp.float32),
        pltpu.VMEM((B, H), jnp.float32),
        pltpu.VMEM((DCH, H), jnp.float32),
        pltpu.VMEM((DCH, H), jnp.float32),
        pltpu.VMEM((DCH, H), jnp.float32),
        pltpu.VMEM((DCH,), jnp.float32),
        pltpu.VMEM((DCH,), jnp.float32),
        pltpu.VMEM((BPT,), jnp.int32),
        pltpu.VMEM((BPT,), jnp.int32),
        pltpu.VMEM((BPT,), jnp.float32),
        pltpu.VMEM((BPT,), jnp.float32),
    ],
)


@jax.jit
def kernel(user_id, item_ids, edge_index, users_emb, items_emb):
    row = edge_index[0].astype(jnp.int32)
    col = edge_index[1].astype(jnp.int32)
    pad = jnp.full((E_PAD - E,), DUMP, dtype=jnp.int32)
    row2d = jnp.concatenate([row, pad]).reshape(E_PAD // B, B)
    col1d = jnp.concatenate([col, pad])

    zpad = jnp.zeros((NP - N_NODES, H), jnp.float32)
    halves = []
    for c in range(NC):
        halves.append(jnp.concatenate([
            users_emb[:, c * H:(c + 1) * H],
            items_emb[:, c * H:(c + 1) * H],
            zpad,
        ], axis=0))
    e0 = jnp.concatenate(halves, axis=0)

    uid = user_id.astype(jnp.int32)
    iid = item_ids.astype(jnp.int32) + NUM_USERS

    out_u, out_i, _t, _p = _sc_call(row2d, col1d, e0, uid, iid)
    return jnp.concatenate([out_u[0], out_u[1], out_i[0], out_i[1]], axis=1)


# speedup vs baseline: 13.3021x; 1.3806x over previous
import jax
import jax.numpy as jnp
from jax import lax
from jax.experimental import pallas as pl
from jax.experimental.pallas import tpu as pltpu
from jax.experimental.pallas import tpu_sc as plsc

NUM_USERS = 10000
NUM_ITEMS = 40000
N_NODES = NUM_USERS + NUM_ITEMS
H = 32
K_LAYERS = 3
BATCH = 4096

NC = 2
NS = 16
NP = 50176
DUMP = N_NODES
NPT = NP // NS
DCH = 64
NDCH = NPT // DCH

E = 800000
E_PAD = 819200
EPT = E_PAD // NS
B = 128
JB = 16
CH = JB * B
NCH = EPT // CH
RPT = EPT // B
RB = 3
BPT = BATCH // NS
OCH = 64


def _sc_body(row2d, col1d, e0, uid, iid, out_u, out_i, t_hbm, p_hbm,
             acc_sp, d2_sp, ring, rowv, ocolv, av, tv, d2c, pbc,
             uidxv, iidxv, puv, piv, sem_g, sem_s):
    c = lax.axis_index("c")
    s = lax.axis_index("s")
    off = c * NP
    nbase0 = s * NPT

    zeros16 = jnp.zeros((16,), jnp.float32)
    ones16 = jnp.ones((16,), jnp.float32)
    lane = lax.iota(jnp.int32, 16)
    lane0 = lane == 0

    def _zero_ring0():
        def _z(i, _):
            ring[0, i, pl.ds(0, 16)] = zeros16
            ring[0, i, pl.ds(16, 16)] = zeros16
            return 0
        lax.fori_loop(0, DCH, _z, 0)

    def _wait_g():
        pltpu.make_async_copy(t_hbm.at[pl.ds(0, B)], ring.at[0], sem_g).wait()

    def _wait_s():
        pltpu.make_async_copy(t_hbm.at[pl.ds(0, B)], ring.at[0], sem_s).wait()

    _zero_ring0()

    def _init_ones(i, _):
        ring[1, i, pl.ds(0, 16)] = ones16
        ring[1, i, pl.ds(16, 16)] = ones16
        return 0
    lax.fori_loop(0, B, _init_ones, 0)

    def _zero_ch(ch, _):
        pltpu.sync_copy(ring.at[0, pl.ds(0, DCH)],
                        acc_sp.at[pl.ds(nbase0 + ch * DCH, DCH)])
        return 0
    lax.fori_loop(0, NDCH, _zero_ch, 0)
    plsc.subcore_barrier()

    def _deg_chunk(m, _):
        pltpu.sync_copy(row2d.at[pl.ds(s * RPT + m * JB, JB)], rowv.at[0])

        def _deg_j(j, _):
            pltpu.async_copy(ring.at[1], acc_sp.at[rowv.at[0, j]], sem_s,
                             add=True)
            return 0
        lax.fori_loop(0, JB, _deg_j, 0)

        def _drain(j, _):
            _wait_s()
            return 0
        lax.fori_loop(0, JB, _drain, 0)
        return 0
    lax.fori_loop(0, NCH, _deg_chunk, 0)
    plsc.subcore_barrier()

    _zero_ring0()

    def _c_chunk(ch, _):
        nb = nbase0 + ch * DCH
        pltpu.sync_copy(acc_sp.at[pl.ds(nb, DCH)], av)
        pltpu.sync_copy(ring.at[0, pl.ds(0, DCH)], acc_sp.at[pl.ds(nb, DCH)])
        pltpu.sync_copy(e0.at[pl.ds(off + nb, DCH)], tv)

        def _row_c(n, _):
            d = av[n, pl.ds(0, 16)]
            gb = jnp.full((16,), nb + n, jnp.int32)
            real = (d > 0.5) & (gb < N_NODES)
            xi = lax.bitcast_convert_type(d, jnp.int32)
            y = lax.bitcast_convert_type(0x5F3759DF - (xi >> 1), jnp.float32)
            hx = 0.5 * d
            y = y * (1.5 - hx * y * y)
            y = y * (1.5 - hx * y * y)
            y = y * (1.5 - hx * y * y)
            d1 = jnp.where(real, y, 0.0)
            tv[n, pl.ds(0, 16)] = tv[n, pl.ds(0, 16)] * d1
            tv[n, pl.ds(16, 16)] = tv[n, pl.ds(16, 16)] * d1
            nloc = jnp.full((16,), n, jnp.int32)
            plsc.store_scatter(d2c, [nloc], d1 * d1, mask=lane0)
            plsc.store_scatter(pbc, [nloc], d * d1, mask=lane0)
            return 0
        lax.fori_loop(0, DCH, _row_c, 0)
        pltpu.sync_copy(tv, t_hbm.at[pl.ds(off + nb, DCH)])
        pltpu.sync_copy(d2c, d2_sp.at[pl.ds(nb, DCH)])
        pltpu.sync_copy(pbc, p_hbm.at[pl.ds(off + nb, DCH)])
        return 0
    lax.fori_loop(0, NDCH, _c_chunk, 0)
    plsc.subcore_barrier()

    pltpu.sync_copy(uid.at[pl.ds(s * BPT, BPT)], uidxv)
    pltpu.sync_copy(iid.at[pl.ds(s * BPT, BPT)], iidxv)

    def _off_idx(q, _):
        uidxv[pl.ds(q * 16, 16)] = uidxv[pl.ds(q * 16, 16)] + off
        iidxv[pl.ds(q * 16, 16)] = iidxv[pl.ds(q * 16, 16)] + off
        return 0
    lax.fori_loop(0, BPT // 16, _off_idx, 0)

    for q in range(BPT // B):
        pltpu.sync_copy(p_hbm.at[uidxv.at[pl.ds(q * B, B)]],
                        puv.at[pl.ds(q * B, B)])
        pltpu.sync_copy(p_hbm.at[iidxv.at[pl.ds(q * B, B)]],
                        piv.at[pl.ds(q * B, B)])
        pltpu.sync_copy(e0.at[uidxv.at[pl.ds(q * B, B)]], ring.at[2])
        pltpu.sync_copy(ring.at[2], out_u.at[c, pl.ds(s * BPT + q * B, B)])
        pltpu.sync_copy(e0.at[iidxv.at[pl.ds(q * B, B)]], ring.at[2])
        pltpu.sync_copy(ring.at[2], out_i.at[c, pl.ds(s * BPT + q * B, B)])

    def _load_chunk(m):
        par = m % 2
        pltpu.sync_copy(row2d.at[pl.ds(s * RPT + m * JB, JB)], rowv.at[par])
        pltpu.sync_copy(col1d.at[pl.ds(s * EPT + m * CH, CH)], ocolv.at[par])

        def _off_col(q, _):
            ocolv[par, pl.ds(q * 16, 16)] = ocolv[par, pl.ds(q * 16, 16)] + off
            return 0
        lax.fori_loop(0, CH // 16, _off_col, 0)

    def _issue_gather(jj):
        m = jj // JB
        pltpu.async_copy(
            t_hbm.at[ocolv.at[m % 2, pl.ds((jj % JB) * B, B)]],
            ring.at[jj % RB], sem_g)

    def _layer(k, _):
        fac = jnp.where(k == K_LAYERS - 1, jnp.float32(0.25), jnp.float32(1.0))

        _load_chunk(0)
        _issue_gather(0)
        _issue_gather(1)

        def _edge_step(jj, _):
            m = jj // JB
            j = jj - m * JB
            _wait_g()
            pltpu.async_copy(ring.at[jj % RB],
                             acc_sp.at[rowv.at[m % 2, j]], sem_s, add=True)

            @pl.when(jj > 0)
            def _():
                _wait_s()

            jn = jj + 2

            @pl.when(jn < RPT)
            def _():
                mn = jn // JB

                @pl.when(jn - mn * JB == 0)
                def _():
                    _load_chunk(mn)
                _issue_gather(jn)
            return 0
        lax.fori_loop(0, RPT, _edge_step, 0)
        _wait_s()
        plsc.subcore_barrier()

        _zero_ring0()

        def _d_chunk(ch, _):
            nb = nbase0 + ch * DCH
            pltpu.sync_copy(acc_sp.at[pl.ds(nb, DCH)], av)
            pltpu.sync_copy(ring.at[0, pl.ds(0, DCH)],
                            acc_sp.at[pl.ds(nb, DCH)])
            pltpu.sync_copy(d2_sp.at[pl.ds(nb, DCH)], d2c)

            def _scale_grp(g, _):
                sv = d2c[pl.ds(g * 16, 16)]
                for l in range(16):
                    n = g * 16 + l
                    sc = sv[l]
                    tv[n, pl.ds(0, 16)] = av[n, pl.ds(0, 16)] * sc
                    tv[n, pl.ds(16, 16)] = av[n, pl.ds(16, 16)] * sc
                return 0
            lax.fori_loop(0, DCH // 16, _scale_grp, 0)
            pltpu.sync_copy(tv, t_hbm.at[pl.ds(off + nb, DCH)])
            return 0
        lax.fori_loop(0, NDCH, _d_chunk, 0)
        plsc.subcore_barrier()

        for q in range(BPT // OCH):
            for which in range(2):
                idxv = uidxv if which == 0 else iidxv
                pv_ref = puv if which == 0 else piv
                out_ref = out_u if which == 0 else out_i
                pltpu.sync_copy(t_hbm.at[idxv.at[pl.ds(q * OCH, OCH)]],
                                ring.at[2, pl.ds(0, OCH)])
                pltpu.sync_copy(out_ref.at[c, pl.ds(s * BPT + q * OCH, OCH)],
                                av)

                def _acc_out(g, _):
                    pv = pv_ref[pl.ds(q * OCH + g * 16, 16)]
                    for l in range(16):
                        r = g * 16 + l
                        pr = pv[l]
                        for hh in (0, 16):
                            x = av[r, pl.ds(hh, 16)] + pr * ring[2, r, pl.ds(hh, 16)]
                            av[r, pl.ds(hh, 16)] = x * fac
                    return 0
                lax.fori_loop(0, OCH // 16, _acc_out, 0)
                pltpu.sync_copy(av,
                                out_ref.at[c, pl.ds(s * BPT + q * OCH, OCH)])
        return 0
    lax.fori_loop(0, K_LAYERS, _layer, 0)


_sc_call = pl.kernel(
    _sc_body,
    out_type=[
        jax.ShapeDtypeStruct((NC, BATCH, H), jnp.float32),
        jax.ShapeDtypeStruct((NC, BATCH, H), jnp.float32),
        jax.ShapeDtypeStruct((NC * NP, H), jnp.float32),
        jax.ShapeDtypeStruct((NC * NP,), jnp.float32),
    ],
    mesh=plsc.VectorSubcoreMesh(core_axis_name="c", subcore_axis_name="s",
                                num_cores=NC, num_subcores=NS),
    compiler_params=pltpu.CompilerParams(use_tc_tiling_on_sc=False,
                                         needs_layout_passes=False),
    scratch_types=[
        pltpu.VMEM_SHARED((NP, H), jnp.float32),
        pltpu.VMEM_SHARED((NP,), jnp.float32),
        pltpu.VMEM((RB, B, H), jnp.float32),
        pltpu.VMEM((2, JB, B), jnp.int32),
        pltpu.VMEM((2, CH), jnp.int32),
        pltpu.VMEM((DCH, H), jnp.float32),
        pltpu.VMEM((DCH, H), jnp.float32),
        pltpu.VMEM((DCH,), jnp.float32),
        pltpu.VMEM((DCH,), jnp.float32),
        pltpu.VMEM((BPT,), jnp.int32),
        pltpu.VMEM((BPT,), jnp.int32),
        pltpu.VMEM((BPT,), jnp.float32),
        pltpu.VMEM((BPT,), jnp.float32),
        pltpu.SemaphoreType.DMA,
        pltpu.SemaphoreType.DMA,
    ],
)


@jax.jit
def kernel(user_id, item_ids, edge_index, users_emb, items_emb):
    row = edge_index[0].astype(jnp.int32)
    col = edge_index[1].astype(jnp.int32)
    pad = jnp.full((E_PAD - E,), DUMP, dtype=jnp.int32)
    row2d = jnp.concatenate([row, pad]).reshape(E_PAD // B, B)
    col1d = jnp.concatenate([col, pad])

    zpad = jnp.zeros((NP - N_NODES, H), jnp.float32)
    halves = []
    for c in range(NC):
        halves.append(jnp.concatenate([
            users_emb[:, c * H:(c + 1) * H],
            items_emb[:, c * H:(c + 1) * H],
            zpad,
        ], axis=0))
    e0 = jnp.concatenate(halves, axis=0)

    uid = user_id.astype(jnp.int32)
    iid = item_ids.astype(jnp.int32) + NUM_USERS

    out_u, out_i, _t, _p = _sc_call(row2d, col1d, e0, uid, iid)
    return jnp.concatenate([out_u[0], out_u[1], out_i[0], out_i[1]], axis=1)


# speedup vs baseline: 13.3084x; 1.0005x over previous
import jax
import jax.numpy as jnp
from jax import lax
from jax.experimental import pallas as pl
from jax.experimental.pallas import tpu as pltpu
from jax.experimental.pallas import tpu_sc as plsc

NUM_USERS = 10000
NUM_ITEMS = 40000
N_NODES = NUM_USERS + NUM_ITEMS
H = 32
K_LAYERS = 3
BATCH = 4096

NC = 2
NS = 16
NP = 50176
DUMP = N_NODES
NPT = NP // NS
DCH = 64
NDCH = NPT // DCH

E = 800000
E_PAD = 819200
EPT = E_PAD // NS
B = 128
JB = 16
CH = JB * B
NCH = EPT // CH
RPT = EPT // B
RB = 3
BPT = BATCH // NS
OCH = 64


def _sc_body(row2d, col1d, e0, uid, iid, out_u, out_i, t_hbm, p_hbm,
             acc_sp, d2_sp, ring, rowv, ocolv, av, tv, d2c, pbc,
             uidxv, iidxv, puv, piv, sem_g, sem_s):
    c = lax.axis_index("c")
    s = lax.axis_index("s")
    off = c * NP
    nbase0 = s * NPT

    zeros16 = jnp.zeros((16,), jnp.float32)
    ones16 = jnp.ones((16,), jnp.float32)
    lane = lax.iota(jnp.int32, 16)
    lane0 = lane == 0

    def _zero_ring0():
        def _z(i, _):
            ring[0, i, pl.ds(0, 16)] = zeros16
            ring[0, i, pl.ds(16, 16)] = zeros16
            return 0
        lax.fori_loop(0, DCH, _z, 0)

    def _wait_g():
        pltpu.make_async_copy(t_hbm.at[pl.ds(0, B)], ring.at[0], sem_g).wait()

    def _wait_s():
        pltpu.make_async_copy(t_hbm.at[pl.ds(0, B)], ring.at[0], sem_s).wait()

    _zero_ring0()

    def _init_ones(i, _):
        ring[1, i, pl.ds(0, 16)] = ones16
        ring[1, i, pl.ds(16, 16)] = ones16
        return 0
    lax.fori_loop(0, B, _init_ones, 0)

    def _zero_ch(ch, _):
        pltpu.sync_copy(ring.at[0, pl.ds(0, DCH)],
                        acc_sp.at[pl.ds(nbase0 + ch * DCH, DCH)])
        return 0
    lax.fori_loop(0, NDCH, _zero_ch, 0)
    plsc.subcore_barrier()

    def _deg_chunk(m, _):
        pltpu.sync_copy(row2d.at[pl.ds(s * RPT + m * JB, JB)], rowv.at[0])

        def _deg_j(j, _):
            pltpu.async_copy(ring.at[1], acc_sp.at[rowv.at[0, j]], sem_s,
                             add=True)
            return 0
        lax.fori_loop(0, JB, _deg_j, 0)

        def _drain(j, _):
            _wait_s()
            return 0
        lax.fori_loop(0, JB, _drain, 0)
        return 0
    with jax.named_scope("phaseB_deg"):
        lax.fori_loop(0, NCH, _deg_chunk, 0)
    plsc.subcore_barrier()

    _zero_ring0()

    def _c_chunk(ch, _):
        nb = nbase0 + ch * DCH
        pltpu.sync_copy(acc_sp.at[pl.ds(nb, DCH)], av)
        pltpu.sync_copy(ring.at[0, pl.ds(0, DCH)], acc_sp.at[pl.ds(nb, DCH)])
        pltpu.sync_copy(e0.at[pl.ds(off + nb, DCH)], tv)

        def _row_c(n, _):
            d = av[n, pl.ds(0, 16)]
            gb = jnp.full((16,), nb + n, jnp.int32)
            real = (d > 0.5) & (gb < N_NODES)
            xi = lax.bitcast_convert_type(d, jnp.int32)
            y = lax.bitcast_convert_type(0x5F3759DF - (xi >> 1), jnp.float32)
            hx = 0.5 * d
            y = y * (1.5 - hx * y * y)
            y = y * (1.5 - hx * y * y)
            y = y * (1.5 - hx * y * y)
            d1 = jnp.where(real, y, 0.0)
            tv[n, pl.ds(0, 16)] = tv[n, pl.ds(0, 16)] * d1
            tv[n, pl.ds(16, 16)] = tv[n, pl.ds(16, 16)] * d1
            nloc = jnp.full((16,), n, jnp.int32)
            plsc.store_scatter(d2c, [nloc], d1 * d1, mask=lane0)
            plsc.store_scatter(pbc, [nloc], d * d1, mask=lane0)
            return 0
        lax.fori_loop(0, DCH, _row_c, 0)
        pltpu.sync_copy(tv, t_hbm.at[pl.ds(off + nb, DCH)])
        pltpu.sync_copy(d2c, d2_sp.at[pl.ds(nb, DCH)])
        pltpu.sync_copy(pbc, p_hbm.at[pl.ds(off + nb, DCH)])
        return 0
    with jax.named_scope("phaseC"):
        lax.fori_loop(0, NDCH, _c_chunk, 0)
    plsc.subcore_barrier()

    pltpu.sync_copy(uid.at[pl.ds(s * BPT, BPT)], uidxv)
    pltpu.sync_copy(iid.at[pl.ds(s * BPT, BPT)], iidxv)

    def _off_idx(q, _):
        uidxv[pl.ds(q * 16, 16)] = uidxv[pl.ds(q * 16, 16)] + off
        iidxv[pl.ds(q * 16, 16)] = iidxv[pl.ds(q * 16, 16)] + off
        return 0
    lax.fori_loop(0, BPT // 16, _off_idx, 0)

    for q in range(BPT // B):
        pltpu.sync_copy(p_hbm.at[uidxv.at[pl.ds(q * B, B)]],
                        puv.at[pl.ds(q * B, B)])
        pltpu.sync_copy(p_hbm.at[iidxv.at[pl.ds(q * B, B)]],
                        piv.at[pl.ds(q * B, B)])
        pltpu.sync_copy(e0.at[uidxv.at[pl.ds(q * B, B)]], ring.at[2])
        pltpu.sync_copy(ring.at[2], out_u.at[c, pl.ds(s * BPT + q * B, B)])
        pltpu.sync_copy(e0.at[iidxv.at[pl.ds(q * B, B)]], ring.at[2])
        pltpu.sync_copy(ring.at[2], out_i.at[c, pl.ds(s * BPT + q * B, B)])

    def _load_chunk(m):
        par = m % 2
        pltpu.sync_copy(row2d.at[pl.ds(s * RPT + m * JB, JB)], rowv.at[par])
        pltpu.sync_copy(col1d.at[pl.ds(s * EPT + m * CH, CH)], ocolv.at[par])

        def _off_col(q, _):
            ocolv[par, pl.ds(q * 16, 16)] = ocolv[par, pl.ds(q * 16, 16)] + off
            return 0
        lax.fori_loop(0, CH // 16, _off_col, 0)

    def _issue_gather(jj):
        m = jj // JB
        pltpu.async_copy(
            t_hbm.at[ocolv.at[m % 2, pl.ds((jj % JB) * B, B)]],
            ring.at[jj % RB], sem_g)

    def _layer(k, _):
        fac = jnp.where(k == K_LAYERS - 1, jnp.float32(0.25), jnp.float32(1.0))

        _load_chunk(0)
        _issue_gather(0)
        _issue_gather(1)

        def _edge_step(jj, _):
            m = jj // JB
            j = jj - m * JB
            _wait_g()
            pltpu.async_copy(ring.at[jj % RB],
                             acc_sp.at[rowv.at[m % 2, j]], sem_s, add=True)

            @pl.when(jj > 0)
            def _():
                _wait_s()

            jn = jj + 2

            @pl.when(jn < RPT)
            def _():
                mn = jn // JB

                @pl.when(jn - mn * JB == 0)
                def _():
                    _load_chunk(mn)
                _issue_gather(jn)
            return 0
        with jax.named_scope("E1_edges"):
            lax.fori_loop(0, RPT, _edge_step, 0)
        _wait_s()
        plsc.subcore_barrier()

        _zero_ring0()

        def _d_chunk(ch, _):
            nb = nbase0 + ch * DCH
            pltpu.sync_copy(acc_sp.at[pl.ds(nb, DCH)], av)
            pltpu.sync_copy(ring.at[0, pl.ds(0, DCH)],
                            acc_sp.at[pl.ds(nb, DCH)])
            pltpu.sync_copy(d2_sp.at[pl.ds(nb, DCH)], d2c)

            def _scale_grp(g, _):
                sv = d2c[pl.ds(g * 16, 16)]
                for l in range(16):
                    n = g * 16 + l
                    sc = sv[l]
                    tv[n, pl.ds(0, 16)] = av[n, pl.ds(0, 16)] * sc
                    tv[n, pl.ds(16, 16)] = av[n, pl.ds(16, 16)] * sc
                return 0
            lax.fori_loop(0, DCH // 16, _scale_grp, 0)
            pltpu.sync_copy(tv, t_hbm.at[pl.ds(off + nb, DCH)])
            return 0
        with jax.named_scope("E2_dense"):
            lax.fori_loop(0, NDCH, _d_chunk, 0)
        plsc.subcore_barrier()

        for q in range(BPT // OCH):
            for which in range(2):
                idxv = uidxv if which == 0 else iidxv
                pv_ref = puv if which == 0 else piv
                out_ref = out_u if which == 0 else out_i
                pltpu.sync_copy(t_hbm.at[idxv.at[pl.ds(q * OCH, OCH)]],
                                ring.at[2, pl.ds(0, OCH)])
                pltpu.sync_copy(out_ref.at[c, pl.ds(s * BPT + q * OCH, OCH)],
                                av)

                def _acc_out(g, _):
                    pv = pv_ref[pl.ds(q * OCH + g * 16, 16)]
                    for l in range(16):
                        r = g * 16 + l
                        pr = pv[l]
                        for hh in (0, 16):
                            x = av[r, pl.ds(hh, 16)] + pr * ring[2, r, pl.ds(hh, 16)]
                            av[r, pl.ds(hh, 16)] = x * fac
                    return 0
                lax.fori_loop(0, OCH // 16, _acc_out, 0)
                pltpu.sync_copy(av,
                                out_ref.at[c, pl.ds(s * BPT + q * OCH, OCH)])
        return 0
    lax.fori_loop(0, K_LAYERS, _layer, 0)


_sc_call = pl.kernel(
    _sc_body,
    out_type=[
        jax.ShapeDtypeStruct((NC, BATCH, H), jnp.float32),
        jax.ShapeDtypeStruct((NC, BATCH, H), jnp.float32),
        jax.ShapeDtypeStruct((NC * NP, H), jnp.float32),
        jax.ShapeDtypeStruct((NC * NP,), jnp.float32),
    ],
    mesh=plsc.VectorSubcoreMesh(core_axis_name="c", subcore_axis_name="s",
                                num_cores=NC, num_subcores=NS),
    compiler_params=pltpu.CompilerParams(use_tc_tiling_on_sc=False,
                                         needs_layout_passes=False),
    scratch_types=[
        pltpu.VMEM_SHARED((NP, H), jnp.float32),
        pltpu.VMEM_SHARED((NP,), jnp.float32),
        pltpu.VMEM((RB, B, H), jnp.float32),
        pltpu.VMEM((2, JB, B), jnp.int32),
        pltpu.VMEM((2, CH), jnp.int32),
        pltpu.VMEM((DCH, H), jnp.float32),
        pltpu.VMEM((DCH, H), jnp.float32),
        pltpu.VMEM((DCH,), jnp.float32),
        pltpu.VMEM((DCH,), jnp.float32),
        pltpu.VMEM((BPT,), jnp.int32),
        pltpu.VMEM((BPT,), jnp.int32),
        pltpu.VMEM((BPT,), jnp.float32),
        pltpu.VMEM((BPT,), jnp.float32),
        pltpu.SemaphoreType.DMA,
        pltpu.SemaphoreType.DMA,
    ],
)


@jax.jit
def kernel(user_id, item_ids, edge_index, users_emb, items_emb):
    row = edge_index[0].astype(jnp.int32)
    col = edge_index[1].astype(jnp.int32)
    pad = jnp.full((E_PAD - E,), DUMP, dtype=jnp.int32)
    row2d = jnp.concatenate([row, pad]).reshape(E_PAD // B, B)
    col1d = jnp.concatenate([col, pad])

    zpad = jnp.zeros((NP - N_NODES, H), jnp.float32)
    halves = []
    for c in range(NC):
        halves.append(jnp.concatenate([
            users_emb[:, c * H:(c + 1) * H],
            items_emb[:, c * H:(c + 1) * H],
            zpad,
        ], axis=0))
    e0 = jnp.concatenate(halves, axis=0)

    uid = user_id.astype(jnp.int32)
    iid = item_ids.astype(jnp.int32) + NUM_USERS

    out_u, out_i, _t, _p = _sc_call(row2d, col1d, e0, uid, iid)
    return jnp.concatenate([out_u[0], out_u[1], out_i[0], out_i[1]], axis=1)


# speedup vs baseline: 13.3150x; 1.0005x over previous
import jax
import jax.numpy as jnp
from jax import lax
from jax.experimental import pallas as pl
from jax.experimental.pallas import tpu as pltpu
from jax.experimental.pallas import tpu_sc as plsc

NUM_USERS = 10000
NUM_ITEMS = 40000
N_NODES = NUM_USERS + NUM_ITEMS
H = 32
K_LAYERS = 3
BATCH = 4096

NC = 2
NS = 16
NP = 50176
DUMP = N_NODES
NPT = NP // NS
DCH = 64
NDCH = NPT // DCH

E = 800000
E_PAD = 819200
EPT = E_PAD // NS
B = 128
JB = 16
CH = JB * B
NCH = EPT // CH
RPT = EPT // B
RB = 3
BPT = BATCH // NS
OCH = 64


def _sc_body(row2d, col1d, e0, uid, iid, out_u, out_i, t_hbm, p_hbm,
             acc_sp, d2_sp, ring, rowv, ocolv, av, tv, d2c, pbc,
             uidxv, iidxv, puv, piv, sem_g, sem_s):
    c = lax.axis_index("c")
    s = lax.axis_index("s")
    off = c * NP
    nbase0 = s * NPT

    zeros16 = jnp.zeros((16,), jnp.float32)
    ones16 = jnp.ones((16,), jnp.float32)
    lane = lax.iota(jnp.int32, 16)
    lane0 = lane == 0

    def _zero_ring0():
        def _z(i, _):
            ring[0, i, pl.ds(0, 16)] = zeros16
            ring[0, i, pl.ds(16, 16)] = zeros16
            return 0
        lax.fori_loop(0, DCH, _z, 0)

    def _wait_g():
        pltpu.make_async_copy(t_hbm.at[pl.ds(0, B)], ring.at[0], sem_g).wait()

    def _wait_s():
        pltpu.make_async_copy(t_hbm.at[pl.ds(0, B)], ring.at[0], sem_s).wait()

    _zero_ring0()

    def _init_ones(i, _):
        ring[1, i, pl.ds(0, 16)] = ones16
        ring[1, i, pl.ds(16, 16)] = ones16
        return 0
    lax.fori_loop(0, B, _init_ones, 0)

    def _zero_ch(ch, _):
        pltpu.sync_copy(ring.at[0, pl.ds(0, DCH)],
                        acc_sp.at[pl.ds(nbase0 + ch * DCH, DCH)])
        return 0
    with jax.named_scope("phaseA"):
        lax.fori_loop(0, NDCH, _zero_ch, 0)
    plsc.subcore_barrier()

    def _deg_chunk(m, _):
        pltpu.sync_copy(row2d.at[pl.ds(s * RPT + m * JB, JB)], rowv.at[0])

        def _deg_j(j, _):
            pltpu.async_copy(ring.at[1], acc_sp.at[rowv.at[0, j]], sem_s,
                             add=True)
            return 0
        lax.fori_loop(0, JB, _deg_j, 0)

        def _drain(j, _):
            _wait_s()
            return 0
        lax.fori_loop(0, JB, _drain, 0)
        return 0
    with jax.named_scope("phaseB_deg"):
        lax.fori_loop(0, NCH, _deg_chunk, 0)
    plsc.subcore_barrier()

    _zero_ring0()

    def _c_chunk(ch, _):
        nb = nbase0 + ch * DCH
        pltpu.sync_copy(acc_sp.at[pl.ds(nb, DCH)], av)
        pltpu.sync_copy(ring.at[0, pl.ds(0, DCH)], acc_sp.at[pl.ds(nb, DCH)])
        pltpu.sync_copy(e0.at[pl.ds(off + nb, DCH)], tv)

        def _row_c(n, _):
            d = av[n, pl.ds(0, 16)]
            gb = jnp.full((16,), nb + n, jnp.int32)
            real = (d > 0.5) & (gb < N_NODES)
            xi = lax.bitcast_convert_type(d, jnp.int32)
            y = lax.bitcast_convert_type(0x5F3759DF - (xi >> 1), jnp.float32)
            hx = 0.5 * d
            y = y * (1.5 - hx * y * y)
            y = y * (1.5 - hx * y * y)
            y = y * (1.5 - hx * y * y)
            d1 = jnp.where(real, y, 0.0)
            tv[n, pl.ds(0, 16)] = tv[n, pl.ds(0, 16)] * d1
            tv[n, pl.ds(16, 16)] = tv[n, pl.ds(16, 16)] * d1
            nloc = jnp.full((16,), n, jnp.int32)
            plsc.store_scatter(d2c, [nloc], d1 * d1, mask=lane0)
            plsc.store_scatter(pbc, [nloc], d * d1, mask=lane0)
            return 0
        lax.fori_loop(0, DCH, _row_c, 0)
        pltpu.sync_copy(tv, t_hbm.at[pl.ds(off + nb, DCH)])
        pltpu.sync_copy(d2c, d2_sp.at[pl.ds(nb, DCH)])
        pltpu.sync_copy(pbc, p_hbm.at[pl.ds(off + nb, DCH)])
        return 0
    with jax.named_scope("phaseC"):
        lax.fori_loop(0, NDCH, _c_chunk, 0)
    plsc.subcore_barrier()

    sD = jax.named_scope("phaseD"); sD.__enter__()
    pltpu.sync_copy(uid.at[pl.ds(s * BPT, BPT)], uidxv)
    pltpu.sync_copy(iid.at[pl.ds(s * BPT, BPT)], iidxv)

    def _off_idx(q, _):
        uidxv[pl.ds(q * 16, 16)] = uidxv[pl.ds(q * 16, 16)] + off
        iidxv[pl.ds(q * 16, 16)] = iidxv[pl.ds(q * 16, 16)] + off
        return 0
    lax.fori_loop(0, BPT // 16, _off_idx, 0)

    for q in range(BPT // B):
        pltpu.sync_copy(p_hbm.at[uidxv.at[pl.ds(q * B, B)]],
                        puv.at[pl.ds(q * B, B)])
        pltpu.sync_copy(p_hbm.at[iidxv.at[pl.ds(q * B, B)]],
                        piv.at[pl.ds(q * B, B)])
        pltpu.sync_copy(e0.at[uidxv.at[pl.ds(q * B, B)]], ring.at[2])
        pltpu.sync_copy(ring.at[2], out_u.at[c, pl.ds(s * BPT + q * B, B)])
        pltpu.sync_copy(e0.at[iidxv.at[pl.ds(q * B, B)]], ring.at[2])
        pltpu.sync_copy(ring.at[2], out_i.at[c, pl.ds(s * BPT + q * B, B)])

    sD.__exit__(None, None, None)

    def _load_chunk(m):
        par = m % 2
        pltpu.sync_copy(row2d.at[pl.ds(s * RPT + m * JB, JB)], rowv.at[par])
        pltpu.sync_copy(col1d.at[pl.ds(s * EPT + m * CH, CH)], ocolv.at[par])

        def _off_col(q, _):
            ocolv[par, pl.ds(q * 16, 16)] = ocolv[par, pl.ds(q * 16, 16)] + off
            return 0
        lax.fori_loop(0, CH // 16, _off_col, 0)

    def _issue_gather(jj):
        m = jj // JB
        pltpu.async_copy(
            t_hbm.at[ocolv.at[m % 2, pl.ds((jj % JB) * B, B)]],
            ring.at[jj % RB], sem_g)

    def _layer(k, _):
        fac = jnp.where(k == K_LAYERS - 1, jnp.float32(0.25), jnp.float32(1.0))

        _load_chunk(0)
        _issue_gather(0)
        _issue_gather(1)

        def _edge_step(jj, _):
            m = jj // JB
            j = jj - m * JB
            _wait_g()
            pltpu.async_copy(ring.at[jj % RB],
                             acc_sp.at[rowv.at[m % 2, j]], sem_s, add=True)

            @pl.when(jj > 0)
            def _():
                _wait_s()

            jn = jj + 2

            @pl.when(jn < RPT)
            def _():
                mn = jn // JB

                @pl.when(jn - mn * JB == 0)
                def _():
                    _load_chunk(mn)
                _issue_gather(jn)
            return 0
        with jax.named_scope("E1_edges"):
            lax.fori_loop(0, RPT, _edge_step, 0)
        _wait_s()
        plsc.subcore_barrier()

        _zero_ring0()

        def _d_chunk(ch, _):
            nb = nbase0 + ch * DCH
            pltpu.sync_copy(acc_sp.at[pl.ds(nb, DCH)], av)
            pltpu.sync_copy(ring.at[0, pl.ds(0, DCH)],
                            acc_sp.at[pl.ds(nb, DCH)])
            pltpu.sync_copy(d2_sp.at[pl.ds(nb, DCH)], d2c)

            def _scale_grp(g, _):
                sv = d2c[pl.ds(g * 16, 16)]
                for l in range(16):
                    n = g * 16 + l
                    sc = sv[l]
                    tv[n, pl.ds(0, 16)] = av[n, pl.ds(0, 16)] * sc
                    tv[n, pl.ds(16, 16)] = av[n, pl.ds(16, 16)] * sc
                return 0
            lax.fori_loop(0, DCH // 16, _scale_grp, 0)
            pltpu.sync_copy(tv, t_hbm.at[pl.ds(off + nb, DCH)])
            return 0
        with jax.named_scope("E2_dense"):
            lax.fori_loop(0, NDCH, _d_chunk, 0)
        plsc.subcore_barrier()

        sE3 = jax.named_scope("E3_out"); sE3.__enter__()
        for q in range(BPT // OCH):
            for which in range(2):
                idxv = uidxv if which == 0 else iidxv
                pv_ref = puv if which == 0 else piv
                out_ref = out_u if which == 0 else out_i
                pltpu.sync_copy(t_hbm.at[idxv.at[pl.ds(q * OCH, OCH)]],
                                ring.at[2, pl.ds(0, OCH)])
                pltpu.sync_copy(out_ref.at[c, pl.ds(s * BPT + q * OCH, OCH)],
                                av)

                def _acc_out(g, _):
                    pv = pv_ref[pl.ds(q * OCH + g * 16, 16)]
                    for l in range(16):
                        r = g * 16 + l
                        pr = pv[l]
                        for hh in (0, 16):
                            x = av[r, pl.ds(hh, 16)] + pr * ring[2, r, pl.ds(hh, 16)]
                            av[r, pl.ds(hh, 16)] = x * fac
                    return 0
                lax.fori_loop(0, OCH // 16, _acc_out, 0)
                pltpu.sync_copy(av,
                                out_ref.at[c, pl.ds(s * BPT + q * OCH, OCH)])
        sE3.__exit__(None, None, None)
        return 0
    lax.fori_loop(0, K_LAYERS, _layer, 0)


_sc_call = pl.kernel(
    _sc_body,
    out_type=[
        jax.ShapeDtypeStruct((NC, BATCH, H), jnp.float32),
        jax.ShapeDtypeStruct((NC, BATCH, H), jnp.float32),
        jax.ShapeDtypeStruct((NC * NP, H), jnp.float32),
        jax.ShapeDtypeStruct((NC * NP,), jnp.float32),
    ],
    mesh=plsc.VectorSubcoreMesh(core_axis_name="c", subcore_axis_name="s",
                                num_cores=NC, num_subcores=NS),
    compiler_params=pltpu.CompilerParams(use_tc_tiling_on_sc=False,
                                         needs_layout_passes=False),
    scratch_types=[
        pltpu.VMEM_SHARED((NP, H), jnp.float32),
        pltpu.VMEM_SHARED((NP,), jnp.float32),
        pltpu.VMEM((RB, B, H), jnp.float32),
        pltpu.VMEM((2, JB, B), jnp.int32),
        pltpu.VMEM((2, CH), jnp.int32),
        pltpu.VMEM((DCH, H), jnp.float32),
        pltpu.VMEM((DCH, H), jnp.float32),
        pltpu.VMEM((DCH,), jnp.float32),
        pltpu.VMEM((DCH,), jnp.float32),
        pltpu.VMEM((BPT,), jnp.int32),
        pltpu.VMEM((BPT,), jnp.int32),
        pltpu.VMEM((BPT,), jnp.float32),
        pltpu.VMEM((BPT,), jnp.float32),
        pltpu.SemaphoreType.DMA,
        pltpu.SemaphoreType.DMA,
    ],
)


@jax.jit
def kernel(user_id, item_ids, edge_index, users_emb, items_emb):
    row = edge_index[0].astype(jnp.int32)
    col = edge_index[1].astype(jnp.int32)
    pad = jnp.full((E_PAD - E,), DUMP, dtype=jnp.int32)
    row2d = jnp.concatenate([row, pad]).reshape(E_PAD // B, B)
    col1d = jnp.concatenate([col, pad])

    zpad = jnp.zeros((NP - N_NODES, H), jnp.float32)
    halves = []
    for c in range(NC):
        halves.append(jnp.concatenate([
            users_emb[:, c * H:(c + 1) * H],
            items_emb[:, c * H:(c + 1) * H],
            zpad,
        ], axis=0))
    e0 = jnp.concatenate(halves, axis=0)

    uid = user_id.astype(jnp.int32)
    iid = item_ids.astype(jnp.int32) + NUM_USERS

    out_u, out_i, _t, _p = _sc_call(row2d, col1d, e0, uid, iid)
    return jnp.concatenate([out_u[0], out_u[1], out_i[0], out_i[1]], axis=1)


# speedup vs baseline: 14.0252x; 1.0533x over previous
import jax
import jax.numpy as jnp
from jax import lax
from jax.experimental import pallas as pl
from jax.experimental.pallas import tpu as pltpu
from jax.experimental.pallas import tpu_sc as plsc

NUM_USERS = 10000
NUM_ITEMS = 40000
N_NODES = NUM_USERS + NUM_ITEMS
H = 32
K_LAYERS = 3
BATCH = 4096

NC = 2
NS = 16
NP = 50176
DUMP = N_NODES
NPT = NP // NS
DCH = 32
NDCH = NPT // DCH

E = 800000
E_PAD = 819200
EPT = E_PAD // NS
B = 128
JB = 16
CH = JB * B
NCH = EPT // CH
RPT = EPT // B
RB = 4
LOOKAHEAD = 3
BPT = BATCH // NS
OCH = 32


def _sc_body(row2d, col1d, e0, uid, iid, out_u, out_i, t_hbm, p_hbm,
             acc_sp, d2_sp, ring, rowv, ocolv, av, d2c, pbc,
             uidxv, iidxv, puv, piv, sem_g, sem_s, sem_i):
    c = lax.axis_index("c")
    s = lax.axis_index("s")
    off = c * NP
    nbase0 = s * NPT

    zeros16 = jnp.zeros((16,), jnp.float32)
    ones16 = jnp.ones((16,), jnp.float32)
    lane = lax.iota(jnp.int32, 16)
    zlane = jnp.zeros((16,), jnp.int32)

    def _zero_ring0():
        def _z(i, _):
            ring[0, i, pl.ds(0, 16)] = zeros16
            ring[0, i, pl.ds(16, 16)] = zeros16
            return 0
        lax.fori_loop(0, DCH, _z, 0)

    def _wait_g():
        pltpu.make_async_copy(t_hbm.at[pl.ds(0, B)], ring.at[0], sem_g).wait()

    def _wait_s():
        pltpu.make_async_copy(t_hbm.at[pl.ds(0, B)], ring.at[0], sem_s).wait()

    def _wait_i():
        pltpu.make_async_copy(row2d.at[pl.ds(0, JB)], rowv.at[0], sem_i).wait()
        pltpu.make_async_copy(col1d.at[pl.ds(0, CH)], ocolv.at[0], sem_i).wait()

    def _off_add(m):
        par = m % 2

        def _oc(q, _):
            ocolv[par, pl.ds(q * 16, 16)] = ocolv[par, pl.ds(q * 16, 16)] + off
            return 0
        lax.fori_loop(0, CH // 16, _oc, 0)

    def _load_chunk_sync(m):
        par = m % 2
        pltpu.sync_copy(row2d.at[pl.ds(s * RPT + m * JB, JB)], rowv.at[par])
        pltpu.sync_copy(col1d.at[pl.ds(s * EPT + m * CH, CH)], ocolv.at[par])
        _off_add(m)

    def _load_chunk_async(m):
        par = m % 2
        pltpu.async_copy(row2d.at[pl.ds(s * RPT + m * JB, JB)], rowv.at[par],
                         sem_i)
        pltpu.async_copy(col1d.at[pl.ds(s * EPT + m * CH, CH)], ocolv.at[par],
                         sem_i)

    def _issue_gather(jj):
        m = jj // JB
        pltpu.async_copy(
            t_hbm.at[ocolv.at[m % 2, pl.ds((jj - m * JB) * B, B)]],
            ring.at[jj % RB], sem_g)

    _zero_ring0()

    def _init_ones(i, _):
        ring[1, i, pl.ds(0, 16)] = ones16
        ring[1, i, pl.ds(16, 16)] = ones16
        return 0
    lax.fori_loop(0, B, _init_ones, 0)

    def _zero_ch(ch, _):
        pltpu.sync_copy(ring.at[0, pl.ds(0, DCH)],
                        acc_sp.at[pl.ds(nbase0 + ch * DCH, DCH)])
        return 0
    lax.fori_loop(0, NDCH, _zero_ch, 0)
    plsc.subcore_barrier()

    _load_chunk_sync(0)

    def _deg_chunk(m, _):
        def _deg_j(j, _):
            pltpu.async_copy(ring.at[1], acc_sp.at[rowv.at[m % 2, j]], sem_s,
                             add=True)
            return 0
        lax.fori_loop(0, JB, _deg_j, 0)

        @pl.when(m + 1 < NCH)
        def _():
            par = (m + 1) % 2
            pltpu.sync_copy(row2d.at[pl.ds(s * RPT + (m + 1) * JB, JB)],
                            rowv.at[par])

        @pl.when(m > 0)
        def _():
            def _drain(j, _):
                _wait_s()
                return 0
            lax.fori_loop(0, JB, _drain, 0)
        return 0
    lax.fori_loop(0, NCH, _deg_chunk, 0)

    def _drain_last(j, _):
        _wait_s()
        return 0
    lax.fori_loop(0, JB, _drain_last, 0)
    plsc.subcore_barrier()

    _zero_ring0()

    def _c_chunk(ch, _):
        nb = nbase0 + ch * DCH
        pltpu.sync_copy(acc_sp.at[pl.ds(nb, DCH)], av)
        pltpu.sync_copy(ring.at[0, pl.ds(0, DCH)], acc_sp.at[pl.ds(nb, DCH)])
        pltpu.sync_copy(e0.at[pl.ds(off + nb, DCH)], ring.at[2, pl.ds(0, DCH)])

        def _grp_c(g, _):
            d = plsc.load_gather(av, [g * 16 + lane, zlane])
            gidx = nb + g * 16 + lane
            real = (d > 0.5) & (gidx < N_NODES)
            xi = lax.bitcast_convert_type(d, jnp.int32)
            y = lax.bitcast_convert_type(0x5F3759DF - (xi >> 1), jnp.float32)
            hx = 0.5 * d
            y = y * (1.5 - hx * y * y)
            y = y * (1.5 - hx * y * y)
            y = y * (1.5 - hx * y * y)
            d1 = jnp.where(real, y, 0.0)
            d2c[pl.ds(g * 16, 16)] = d1 * d1
            pbc[pl.ds(g * 16, 16)] = d * d1
            for l in range(16):
                n = g * 16 + l
                sc = d1[l]
                ring[2, n, pl.ds(0, 16)] = ring[2, n, pl.ds(0, 16)] * sc
                ring[2, n, pl.ds(16, 16)] = ring[2, n, pl.ds(16, 16)] * sc
            return 0
        lax.fori_loop(0, DCH // 16, _grp_c, 0)
        pltpu.sync_copy(ring.at[2, pl.ds(0, DCH)],
                        t_hbm.at[pl.ds(off + nb, DCH)])
        pltpu.sync_copy(d2c, d2_sp.at[pl.ds(nb, DCH)])
        pltpu.sync_copy(pbc, p_hbm.at[pl.ds(off + nb, DCH)])
        return 0
    lax.fori_loop(0, NDCH, _c_chunk, 0)
    plsc.subcore_barrier()

    pltpu.sync_copy(uid.at[pl.ds(s * BPT, BPT)], uidxv)
    pltpu.sync_copy(iid.at[pl.ds(s * BPT, BPT)], iidxv)

    def _off_idx(q, _):
        uidxv[pl.ds(q * 16, 16)] = uidxv[pl.ds(q * 16, 16)] + off
        iidxv[pl.ds(q * 16, 16)] = iidxv[pl.ds(q * 16, 16)] + off
        return 0
    lax.fori_loop(0, BPT // 16, _off_idx, 0)

    for q in range(BPT // B):
        pltpu.sync_copy(p_hbm.at[uidxv.at[pl.ds(q * B, B)]],
                        puv.at[pl.ds(q * B, B)])
        pltpu.sync_copy(p_hbm.at[iidxv.at[pl.ds(q * B, B)]],
                        piv.at[pl.ds(q * B, B)])
        pltpu.sync_copy(e0.at[uidxv.at[pl.ds(q * B, B)]], ring.at[2])
        pltpu.sync_copy(ring.at[2], out_u.at[c, pl.ds(s * BPT + q * B, B)])
        pltpu.sync_copy(e0.at[iidxv.at[pl.ds(q * B, B)]], ring.at[2])
        pltpu.sync_copy(ring.at[2], out_i.at[c, pl.ds(s * BPT + q * B, B)])

    def _layer(k, _):
        fac = jnp.where(k == K_LAYERS - 1, jnp.float32(0.25), jnp.float32(1.0))

        _load_chunk_sync(0)
        _issue_gather(0)
        _issue_gather(1)
        _issue_gather(2)

        def _edge_step(jj, _):
            m = jj // JB
            j = jj - m * JB
            _wait_g()
            pltpu.async_copy(ring.at[jj % RB],
                             acc_sp.at[rowv.at[m % 2, j]], sem_s, add=True)

            @pl.when(jj > 0)
            def _():
                _wait_s()

            jn = jj + LOOKAHEAD

            @pl.when(jn < RPT)
            def _():
                mn = jn // JB
                jjn = jn - mn * JB

                @pl.when(jjn == 0)
                def _():
                    _wait_i()
                    _off_add(mn)

                @pl.when(jnp.logical_and(jjn == 5, mn + 1 < NCH))
                def _():
                    _load_chunk_async(mn + 1)
                _issue_gather(jn)
            return 0
        lax.fori_loop(0, RPT, _edge_step, 0)
        _wait_s()
        plsc.subcore_barrier()

        _zero_ring0()

        def _d_chunk(ch, _):
            nb = nbase0 + ch * DCH
            pltpu.sync_copy(acc_sp.at[pl.ds(nb, DCH)], av)
            pltpu.sync_copy(ring.at[0, pl.ds(0, DCH)],
                            acc_sp.at[pl.ds(nb, DCH)])
            pltpu.sync_copy(d2_sp.at[pl.ds(nb, DCH)], d2c)

            def _scale_grp(g, _):
                sv = d2c[pl.ds(g * 16, 16)]
                for l in range(16):
                    n = g * 16 + l
                    sc = sv[l]
                    av[n, pl.ds(0, 16)] = av[n, pl.ds(0, 16)] * sc
                    av[n, pl.ds(16, 16)] = av[n, pl.ds(16, 16)] * sc
                return 0
            lax.fori_loop(0, DCH // 16, _scale_grp, 0)
            pltpu.sync_copy(av, t_hbm.at[pl.ds(off + nb, DCH)])
            return 0
        lax.fori_loop(0, NDCH, _d_chunk, 0)
        plsc.subcore_barrier()

        for q in range(BPT // OCH):
            for which in range(2):
                idxv = uidxv if which == 0 else iidxv
                pv_ref = puv if which == 0 else piv
                out_ref = out_u if which == 0 else out_i
                pltpu.sync_copy(t_hbm.at[idxv.at[pl.ds(q * OCH, OCH)]],
                                ring.at[2, pl.ds(0, OCH)])
                pltpu.sync_copy(out_ref.at[c, pl.ds(s * BPT + q * OCH, OCH)],
                                av)

                def _acc_out(g, _):
                    pv = pv_ref[pl.ds(q * OCH + g * 16, 16)]
                    for l in range(16):
                        r = g * 16 + l
                        pr = pv[l]
                        for hh in (0, 16):
                            x = av[r, pl.ds(hh, 16)] + pr * ring[2, r, pl.ds(hh, 16)]
                            av[r, pl.ds(hh, 16)] = x * fac
                    return 0
                lax.fori_loop(0, OCH // 16, _acc_out, 0)
                pltpu.sync_copy(av,
                                out_ref.at[c, pl.ds(s * BPT + q * OCH, OCH)])
        return 0
    lax.fori_loop(0, K_LAYERS, _layer, 0)


_sc_call = pl.kernel(
    _sc_body,
    out_type=[
        jax.ShapeDtypeStruct((NC, BATCH, H), jnp.float32),
        jax.ShapeDtypeStruct((NC, BATCH, H), jnp.float32),
        jax.ShapeDtypeStruct((NC * NP, H), jnp.float32),
        jax.ShapeDtypeStruct((NC * NP,), jnp.float32),
    ],
    mesh=plsc.VectorSubcoreMesh(core_axis_name="c", subcore_axis_name="s",
                                num_cores=NC, num_subcores=NS),
    compiler_params=pltpu.CompilerParams(use_tc_tiling_on_sc=False,
                                         needs_layout_passes=False),
    scratch_types=[
        pltpu.VMEM_SHARED((NP, H), jnp.float32),
        pltpu.VMEM_SHARED((NP,), jnp.float32),
        pltpu.VMEM((RB, B, H), jnp.float32),
        pltpu.VMEM((2, JB, B), jnp.int32),
        pltpu.VMEM((2, CH), jnp.int32),
        pltpu.VMEM((DCH, H), jnp.float32),
        pltpu.VMEM((DCH,), jnp.float32),
        pltpu.VMEM((DCH,), jnp.float32),
        pltpu.VMEM((BPT,), jnp.int32),
        pltpu.VMEM((BPT,), jnp.int32),
        pltpu.VMEM((BPT,), jnp.float32),
        pltpu.VMEM((BPT,), jnp.float32),
        pltpu.SemaphoreType.DMA,
        pltpu.SemaphoreType.DMA,
        pltpu.SemaphoreType.DMA,
    ],
)


@jax.jit
def kernel(user_id, item_ids, edge_index, users_emb, items_emb):
    row = edge_index[0].astype(jnp.int32)
    col = edge_index[1].astype(jnp.int32)
    pad = jnp.full((E_PAD - E,), DUMP, dtype=jnp.int32)
    row2d = jnp.concatenate([row, pad]).reshape(E_PAD // B, B)
    col1d = jnp.concatenate([col, pad])

    zpad = jnp.zeros((NP - N_NODES, H), jnp.float32)
    halves = []
    for c in range(NC):
        halves.append(jnp.concatenate([
            users_emb[:, c * H:(c + 1) * H],
            items_emb[:, c * H:(c + 1) * H],
            zpad,
        ], axis=0))
    e0 = jnp.concatenate(halves, axis=0)

    uid = user_id.astype(jnp.int32)
    iid = item_ids.astype(jnp.int32) + NUM_USERS

    out_u, out_i, _t, _p = _sc_call(row2d, col1d, e0, uid, iid)
    return jnp.concatenate([out_u[0], out_u[1], out_i[0], out_i[1]], axis=1)


# speedup vs baseline: 19.1233x; 1.3635x over previous
import jax
import jax.numpy as jnp
from jax import lax
from jax.experimental import pallas as pl
from jax.experimental.pallas import tpu as pltpu
from jax.experimental.pallas import tpu_sc as plsc

NUM_USERS = 10000
NUM_ITEMS = 40000
N_NODES = NUM_USERS + NUM_ITEMS
H = 32
K_LAYERS = 3
BATCH = 4096

NC = 2
NS = 16
NP = 50176
DUMP = N_NODES
NPT = NP // NS
DCH = 32
NDCH = NPT // DCH

E = 800000
E_PAD = 819200
EPT = E_PAD // NS
B = 128
JB = 16
CH = JB * B
NCH = EPT // CH
RPT = EPT // B
RB = 8
LOOKAHEAD = 6
BPT = BATCH // NS
OCH = 32

F32 = jnp.float32
BF16 = jnp.bfloat16
PK = plsc.PackFormat.INTERLEAVED


def _sc_body(row2d, col1d, e0, uid, iid, out_u, out_i, t_hbm, p_hbm,
             acc_sp, deg_sp, d2_sp, ring, rowv, ocolv, abf, af32,
             degc, d2c, pbc, znpt, onesv, uidxv, iidxv, puv, piv,
             sem_g, sem_s, sem_i):
    c = lax.axis_index("c")
    s = lax.axis_index("s")
    off = c * NP
    nbase0 = s * NPT

    zeros16 = jnp.zeros((16,), F32)
    zeros32b = jnp.zeros((32,), BF16)
    ones16 = jnp.ones((16,), F32)
    lane = lax.iota(jnp.int32, 16)

    def _zero_ring0():
        def _z(i, _):
            ring[0, i, pl.ds(0, 32)] = zeros32b
            return 0
        lax.fori_loop(0, DCH, _z, 0)

    def _wait_g():
        pltpu.make_async_copy(t_hbm.at[pl.ds(0, B)], ring.at[0], sem_g).wait()

    def _wait_s():
        pltpu.make_async_copy(t_hbm.at[pl.ds(0, B)], ring.at[0], sem_s).wait()

    def _wait_deg():
        pltpu.make_async_copy(p_hbm.at[pl.ds(0, B)], onesv, sem_s).wait()

    def _wait_i():
        pltpu.make_async_copy(row2d.at[pl.ds(0, JB)], rowv.at[0], sem_i).wait()
        pltpu.make_async_copy(col1d.at[pl.ds(0, CH)], ocolv.at[0], sem_i).wait()

    def _off_add(m):
        par = m % 2

        def _oc(q, _):
            ocolv[par, pl.ds(q * 16, 16)] = ocolv[par, pl.ds(q * 16, 16)] + off
            return 0
        lax.fori_loop(0, CH // 16, _oc, 0)

    def _load_chunk_sync(m):
        par = m % 2
        pltpu.sync_copy(row2d.at[pl.ds(s * RPT + m * JB, JB)], rowv.at[par])
        pltpu.sync_copy(col1d.at[pl.ds(s * EPT + m * CH, CH)], ocolv.at[par])
        _off_add(m)

    def _load_chunk_async(m):
        par = m % 2
        pltpu.async_copy(row2d.at[pl.ds(s * RPT + m * JB, JB)], rowv.at[par],
                         sem_i)
        pltpu.async_copy(col1d.at[pl.ds(s * EPT + m * CH, CH)], ocolv.at[par],
                         sem_i)

    def _issue_gather(jj):
        m = jj // JB
        pltpu.async_copy(
            t_hbm.at[ocolv.at[m % 2, pl.ds((jj - m * JB) * B, B)]],
            ring.at[jj % RB], sem_g)

    _zero_ring0()

    def _init_ones(q, _):
        onesv[pl.ds(q * 16, 16)] = ones16
        return 0
    lax.fori_loop(0, B // 16, _init_ones, 0)

    def _init_znpt(q, _):
        znpt[pl.ds(q * 16, 16)] = zeros16
        return 0
    lax.fori_loop(0, NPT // 16, _init_znpt, 0)

    def _zero_ch(ch, _):
        pltpu.sync_copy(ring.at[0, pl.ds(0, DCH)],
                        acc_sp.at[pl.ds(nbase0 + ch * DCH, DCH)])
        return 0
    lax.fori_loop(0, NDCH, _zero_ch, 0)
    pltpu.sync_copy(znpt, deg_sp.at[pl.ds(nbase0, NPT)])
    plsc.subcore_barrier()

    _load_chunk_sync(0)

    def _deg_chunk(m, _):
        def _deg_j(j, _):
            pltpu.async_copy(onesv, deg_sp.at[rowv.at[m % 2, j]], sem_s,
                             add=True)
            return 0
        lax.fori_loop(0, JB, _deg_j, 0)

        @pl.when(m + 1 < NCH)
        def _():
            par = (m + 1) % 2
            pltpu.sync_copy(row2d.at[pl.ds(s * RPT + (m + 1) * JB, JB)],
                            rowv.at[par])

        @pl.when(m > 0)
        def _():
            def _drain(j, _):
                _wait_deg()
                return 0
            lax.fori_loop(0, JB, _drain, 0)
        return 0
    lax.fori_loop(0, NCH, _deg_chunk, 0)

    def _drain_last(j, _):
        _wait_deg()
        return 0
    lax.fori_loop(0, JB, _drain_last, 0)
    plsc.subcore_barrier()

    def _c_chunk(ch, _):
        nb = nbase0 + ch * DCH
        pltpu.sync_copy(deg_sp.at[pl.ds(nb, DCH)], degc)
        pltpu.sync_copy(e0.at[pl.ds(off + nb, DCH)], af32)

        def _grp_c(g, _):
            d = degc[pl.ds(g * 16, 16)]
            gidx = nb + g * 16 + lane
            real = (d > 0.5) & (gidx < N_NODES)
            xi = lax.bitcast_convert_type(d, jnp.int32)
            y = lax.bitcast_convert_type(0x5F3759DF - (xi >> 1), F32)
            hx = 0.5 * d
            y = y * (1.5 - hx * y * y)
            y = y * (1.5 - hx * y * y)
            y = y * (1.5 - hx * y * y)
            d1 = jnp.where(real, y, 0.0)
            d2c[pl.ds(g * 16, 16)] = d1 * d1
            pbc[pl.ds(g * 16, 16)] = d * d1
            for l in range(16):
                n = g * 16 + l
                sc = d1[l]
                a = af32[n, pl.ds(0, 16)] * sc
                b = af32[n, pl.ds(16, 16)] * sc
                abf[n, pl.ds(0, 32)] = plsc.pack(a, b, format=PK)
            return 0
        lax.fori_loop(0, DCH // 16, _grp_c, 0)
        pltpu.sync_copy(abf, t_hbm.at[pl.ds(off + nb, DCH)])
        pltpu.sync_copy(d2c, d2_sp.at[pl.ds(nb, DCH)])
        pltpu.sync_copy(pbc, p_hbm.at[pl.ds(off + nb, DCH)])
        return 0
    lax.fori_loop(0, NDCH, _c_chunk, 0)
    plsc.subcore_barrier()

    pltpu.sync_copy(uid.at[pl.ds(s * BPT, BPT)], uidxv)
    pltpu.sync_copy(iid.at[pl.ds(s * BPT, BPT)], iidxv)

    def _off_idx(q, _):
        uidxv[pl.ds(q * 16, 16)] = uidxv[pl.ds(q * 16, 16)] + off
        iidxv[pl.ds(q * 16, 16)] = iidxv[pl.ds(q * 16, 16)] + off
        return 0
    lax.fori_loop(0, BPT // 16, _off_idx, 0)

    for q in range(BPT // B):
        pltpu.sync_copy(p_hbm.at[uidxv.at[pl.ds(q * B, B)]],
                        puv.at[pl.ds(q * B, B)])
        pltpu.sync_copy(p_hbm.at[iidxv.at[pl.ds(q * B, B)]],
                        piv.at[pl.ds(q * B, B)])
    for q in range(BPT // OCH):
        pltpu.sync_copy(e0.at[uidxv.at[pl.ds(q * OCH, OCH)]], af32)
        pltpu.sync_copy(af32, out_u.at[c, pl.ds(s * BPT + q * OCH, OCH)])
        pltpu.sync_copy(e0.at[iidxv.at[pl.ds(q * OCH, OCH)]], af32)
        pltpu.sync_copy(af32, out_i.at[c, pl.ds(s * BPT + q * OCH, OCH)])

    def _layer(k, _):
        fac = jnp.where(k == K_LAYERS - 1, jnp.float32(0.25), jnp.float32(1.0))

        _load_chunk_sync(0)
        for w in range(LOOKAHEAD):
            _issue_gather(w)

        def _edge_step(jj, _):
            m = jj // JB
            j = jj - m * JB
            _wait_g()
            pltpu.async_copy(ring.at[jj % RB],
                             acc_sp.at[rowv.at[m % 2, j]], sem_s, add=True)

            @pl.when(jj > 0)
            def _():
                _wait_s()

            jn = jj + LOOKAHEAD

            @pl.when(jn < RPT)
            def _():
                mn = jn // JB
                jjn = jn - mn * JB

                @pl.when(jjn == 0)
                def _():
                    _wait_i()
                    _off_add(mn)

                @pl.when(jnp.logical_and(jjn == 8, mn + 1 < NCH))
                def _():
                    _load_chunk_async(mn + 1)
                _issue_gather(jn)
            return 0
        lax.fori_loop(0, RPT, _edge_step, 0)
        _wait_s()
        plsc.subcore_barrier()

        _zero_ring0()

        def _d_chunk(ch, _):
            nb = nbase0 + ch * DCH
            pltpu.sync_copy(acc_sp.at[pl.ds(nb, DCH)], abf)
            pltpu.sync_copy(ring.at[0, pl.ds(0, DCH)],
                            acc_sp.at[pl.ds(nb, DCH)])
            pltpu.sync_copy(d2_sp.at[pl.ds(nb, DCH)], d2c)

            def _scale_grp(g, _):
                sv = d2c[pl.ds(g * 16, 16)]
                for l in range(16):
                    n = g * 16 + l
                    sc = sv[l]
                    a, b = plsc.unpack(abf[n, pl.ds(0, 32)], format=PK)
                    abf[n, pl.ds(0, 32)] = plsc.pack(a * sc, b * sc, format=PK)
                return 0
            lax.fori_loop(0, DCH // 16, _scale_grp, 0)
            pltpu.sync_copy(abf, t_hbm.at[pl.ds(off + nb, DCH)])
            return 0
        lax.fori_loop(0, NDCH, _d_chunk, 0)
        plsc.subcore_barrier()

        for q in range(BPT // OCH):
            for which in range(2):
                idxv = uidxv if which == 0 else iidxv
                pv_ref = puv if which == 0 else piv
                out_ref = out_u if which == 0 else out_i
                pltpu.sync_copy(t_hbm.at[idxv.at[pl.ds(q * OCH, OCH)]],
                                ring.at[2, pl.ds(0, OCH)])
                pltpu.sync_copy(out_ref.at[c, pl.ds(s * BPT + q * OCH, OCH)],
                                af32)

                def _acc_out(g, _):
                    pv = pv_ref[pl.ds(q * OCH + g * 16, 16)]
                    for l in range(16):
                        r = g * 16 + l
                        pr = pv[l]
                        a, b = plsc.unpack(ring[2, r, pl.ds(0, 32)], format=PK)
                        af32[r, pl.ds(0, 16)] = (af32[r, pl.ds(0, 16)] + pr * a) * fac
                        af32[r, pl.ds(16, 16)] = (af32[r, pl.ds(16, 16)] + pr * b) * fac
                    return 0
                lax.fori_loop(0, OCH // 16, _acc_out, 0)
                pltpu.sync_copy(af32,
                                out_ref.at[c, pl.ds(s * BPT + q * OCH, OCH)])
        return 0
    lax.fori_loop(0, K_LAYERS, _layer, 0)


_sc_call = pl.kernel(
    _sc_body,
    out_type=[
        jax.ShapeDtypeStruct((NC, BATCH, H), jnp.float32),
        jax.ShapeDtypeStruct((NC, BATCH, H), jnp.float32),
        jax.ShapeDtypeStruct((NC * NP, H), BF16),
        jax.ShapeDtypeStruct((NC * NP,), jnp.float32),
    ],
    mesh=plsc.VectorSubcoreMesh(core_axis_name="c", subcore_axis_name="s",
                                num_cores=NC, num_subcores=NS),
    compiler_params=pltpu.CompilerParams(use_tc_tiling_on_sc=False,
                                         needs_layout_passes=False),
    scratch_types=[
        pltpu.VMEM_SHARED((NP, H), BF16),
        pltpu.VMEM_SHARED((NP,), jnp.float32),
        pltpu.VMEM_SHARED((NP,), jnp.float32),
        pltpu.VMEM((RB, B, H), BF16),
        pltpu.VMEM((2, JB, B), jnp.int32),
        pltpu.VMEM((2, CH), jnp.int32),
        pltpu.VMEM((DCH, H), BF16),
        pltpu.VMEM((OCH, H), jnp.float32),
        pltpu.VMEM((DCH,), jnp.float32),
        pltpu.VMEM((DCH,), jnp.float32),
        pltpu.VMEM((DCH,), jnp.float32),
        pltpu.VMEM((NPT,), jnp.float32),
        pltpu.VMEM((B,), jnp.float32),
        pltpu.VMEM((BPT,), jnp.int32),
        pltpu.VMEM((BPT,), jnp.int32),
        pltpu.VMEM((BPT,), jnp.float32),
        pltpu.VMEM((BPT,), jnp.float32),
        pltpu.SemaphoreType.DMA,
        pltpu.SemaphoreType.DMA,
        pltpu.SemaphoreType.DMA,
    ],
)


@jax.jit
def kernel(user_id, item_ids, edge_index, users_emb, items_emb):
    row = edge_index[0].astype(jnp.int32)
    col = edge_index[1].astype(jnp.int32)
    pad = jnp.full((E_PAD - E,), DUMP, dtype=jnp.int32)
    row2d = jnp.concatenate([row, pad]).reshape(E_PAD // B, B)
    col1d = jnp.concatenate([col, pad])

    zpad = jnp.zeros((NP - N_NODES, H), jnp.float32)
    halves = []
    for c in range(NC):
        halves.append(jnp.concatenate([
            users_emb[:, c * H:(c + 1) * H],
            items_emb[:, c * H:(c + 1) * H],
            zpad,
        ], axis=0))
    e0 = jnp.concatenate(halves, axis=0)

    uid = user_id.astype(jnp.int32)
    iid = item_ids.astype(jnp.int32) + NUM_USERS

    out_u, out_i, _t, _p = _sc_call(row2d, col1d, e0, uid, iid)
    return jnp.concatenate([out_u[0], out_u[1], out_i[0], out_i[1]], axis=1)


# speedup vs baseline: 19.7312x; 1.0318x over previous
import jax
import jax.numpy as jnp
from jax import lax
from jax.experimental import pallas as pl
from jax.experimental.pallas import tpu as pltpu
from jax.experimental.pallas import tpu_sc as plsc

NUM_USERS = 10000
NUM_ITEMS = 40000
N_NODES = NUM_USERS + NUM_ITEMS
H = 32
K_LAYERS = 3
BATCH = 4096

NC = 2
NS = 16
NP = 50176
DUMP = N_NODES
NPT = NP // NS
DCH = 32
NDCH = NPT // DCH

E = 800000
E_PAD = 819200
EPT = E_PAD // NS
B = 128
JB = 16
CH = JB * B
NCH = EPT // CH
RPT = EPT // B
RB = 12
LOOKAHEAD = 10
BPT = BATCH // NS
OCH = 32

F32 = jnp.float32
BF16 = jnp.bfloat16
PK = plsc.PackFormat.INTERLEAVED


def _sc_body(row2d, col1d, e0, uid, iid, out_u, out_i, t_hbm, p_hbm,
             acc_sp, deg_sp, d2_sp, ring, rowv, ocolv, abf, af32,
             degc, d2c, pbc, znpt, onesv, uidxv, iidxv, puv, piv,
             sem_g, sem_s, sem_i):
    c = lax.axis_index("c")
    s = lax.axis_index("s")
    off = c * NP
    nbase0 = s * NPT

    zeros16 = jnp.zeros((16,), F32)
    zeros32b = jnp.zeros((32,), BF16)
    ones16 = jnp.ones((16,), F32)
    lane = lax.iota(jnp.int32, 16)

    def _zero_ring0():
        def _z(i, _):
            ring[0, i, pl.ds(0, 32)] = zeros32b
            return 0
        lax.fori_loop(0, DCH, _z, 0)

    def _wait_g():
        pltpu.make_async_copy(t_hbm.at[pl.ds(0, B)], ring.at[0], sem_g).wait()

    def _wait_s():
        pltpu.make_async_copy(t_hbm.at[pl.ds(0, B)], ring.at[0], sem_s).wait()

    def _wait_deg():
        pltpu.make_async_copy(p_hbm.at[pl.ds(0, B)], onesv, sem_s).wait()

    def _wait_i():
        pltpu.make_async_copy(row2d.at[pl.ds(0, JB)], rowv.at[0], sem_i).wait()
        pltpu.make_async_copy(col1d.at[pl.ds(0, CH)], ocolv.at[0], sem_i).wait()

    def _off_add(m):
        par = m % 2

        def _oc(q, _):
            ocolv[par, pl.ds(q * 16, 16)] = ocolv[par, pl.ds(q * 16, 16)] + off
            return 0
        lax.fori_loop(0, CH // 16, _oc, 0)

    def _load_chunk_sync(m):
        par = m % 2
        pltpu.sync_copy(row2d.at[pl.ds(s * RPT + m * JB, JB)], rowv.at[par])
        pltpu.sync_copy(col1d.at[pl.ds(s * EPT + m * CH, CH)], ocolv.at[par])
        _off_add(m)

    def _load_chunk_async(m):
        par = m % 2
        pltpu.async_copy(row2d.at[pl.ds(s * RPT + m * JB, JB)], rowv.at[par],
                         sem_i)
        pltpu.async_copy(col1d.at[pl.ds(s * EPT + m * CH, CH)], ocolv.at[par],
                         sem_i)

    def _issue_gather(jj):
        m = jj // JB
        pltpu.async_copy(
            t_hbm.at[ocolv.at[m % 2, pl.ds((jj - m * JB) * B, B)]],
            ring.at[jj % RB], sem_g)

    _zero_ring0()

    def _init_ones(q, _):
        onesv[pl.ds(q * 16, 16)] = ones16
        return 0
    lax.fori_loop(0, B // 16, _init_ones, 0)

    def _init_znpt(q, _):
        znpt[pl.ds(q * 16, 16)] = zeros16
        return 0
    lax.fori_loop(0, NPT // 16, _init_znpt, 0)

    def _zero_ch(ch, _):
        pltpu.sync_copy(ring.at[0, pl.ds(0, DCH)],
                        acc_sp.at[pl.ds(nbase0 + ch * DCH, DCH)])
        return 0
    lax.fori_loop(0, NDCH, _zero_ch, 0)
    pltpu.sync_copy(znpt, deg_sp.at[pl.ds(nbase0, NPT)])
    plsc.subcore_barrier()

    _load_chunk_sync(0)

    def _deg_chunk(m, _):
        def _deg_j(j, _):
            pltpu.async_copy(onesv, deg_sp.at[rowv.at[m % 2, j]], sem_s,
                             add=True)
            return 0
        lax.fori_loop(0, JB, _deg_j, 0)

        @pl.when(m + 1 < NCH)
        def _():
            par = (m + 1) % 2
            pltpu.sync_copy(row2d.at[pl.ds(s * RPT + (m + 1) * JB, JB)],
                            rowv.at[par])

        @pl.when(m > 0)
        def _():
            def _drain(j, _):
                _wait_deg()
                return 0
            lax.fori_loop(0, JB, _drain, 0)
        return 0
    lax.fori_loop(0, NCH, _deg_chunk, 0)

    def _drain_last(j, _):
        _wait_deg()
        return 0
    lax.fori_loop(0, JB, _drain_last, 0)
    plsc.subcore_barrier()

    def _c_chunk(ch, _):
        nb = nbase0 + ch * DCH
        pltpu.sync_copy(deg_sp.at[pl.ds(nb, DCH)], degc)
        pltpu.sync_copy(e0.at[pl.ds(off + nb, DCH)], af32)

        def _grp_c(g, _):
            d = degc[pl.ds(g * 16, 16)]
            gidx = nb + g * 16 + lane
            real = (d > 0.5) & (gidx < N_NODES)
            xi = lax.bitcast_convert_type(d, jnp.int32)
            y = lax.bitcast_convert_type(0x5F3759DF - (xi >> 1), F32)
            hx = 0.5 * d
            y = y * (1.5 - hx * y * y)
            y = y * (1.5 - hx * y * y)
            y = y * (1.5 - hx * y * y)
            d1 = jnp.where(real, y, 0.0)
            d2c[pl.ds(g * 16, 16)] = d1 * d1
            pbc[pl.ds(g * 16, 16)] = d * d1
            for l in range(16):
                n = g * 16 + l
                sc = d1[l]
                a = af32[n, pl.ds(0, 16)] * sc
                b = af32[n, pl.ds(16, 16)] * sc
                abf[n, pl.ds(0, 32)] = plsc.pack(a, b, format=PK)
            return 0
        lax.fori_loop(0, DCH // 16, _grp_c, 0)
        pltpu.sync_copy(abf, t_hbm.at[pl.ds(off + nb, DCH)])
        pltpu.sync_copy(d2c, d2_sp.at[pl.ds(nb, DCH)])
        pltpu.sync_copy(pbc, p_hbm.at[pl.ds(off + nb, DCH)])
        return 0
    lax.fori_loop(0, NDCH, _c_chunk, 0)
    plsc.subcore_barrier()

    pltpu.sync_copy(uid.at[pl.ds(s * BPT, BPT)], uidxv)
    pltpu.sync_copy(iid.at[pl.ds(s * BPT, BPT)], iidxv)

    def _off_idx(q, _):
        uidxv[pl.ds(q * 16, 16)] = uidxv[pl.ds(q * 16, 16)] + off
        iidxv[pl.ds(q * 16, 16)] = iidxv[pl.ds(q * 16, 16)] + off
        return 0
    lax.fori_loop(0, BPT // 16, _off_idx, 0)

    for q in range(BPT // B):
        pltpu.sync_copy(p_hbm.at[uidxv.at[pl.ds(q * B, B)]],
                        puv.at[pl.ds(q * B, B)])
        pltpu.sync_copy(p_hbm.at[iidxv.at[pl.ds(q * B, B)]],
                        piv.at[pl.ds(q * B, B)])
    for q in range(BPT // OCH):
        pltpu.sync_copy(e0.at[uidxv.at[pl.ds(q * OCH, OCH)]], af32)
        pltpu.sync_copy(af32, out_u.at[c, pl.ds(s * BPT + q * OCH, OCH)])
        pltpu.sync_copy(e0.at[iidxv.at[pl.ds(q * OCH, OCH)]], af32)
        pltpu.sync_copy(af32, out_i.at[c, pl.ds(s * BPT + q * OCH, OCH)])

    def _layer(k, _):
        fac = jnp.where(k == K_LAYERS - 1, jnp.float32(0.25), jnp.float32(1.0))

        _load_chunk_sync(0)
        for w in range(LOOKAHEAD):
            _issue_gather(w)

        def _edge_step(jj, _):
            m = jj // JB
            j = jj - m * JB
            _wait_g()
            pltpu.async_copy(ring.at[jj % RB],
                             acc_sp.at[rowv.at[m % 2, j]], sem_s, add=True)

            @pl.when(jj > 0)
            def _():
                _wait_s()

            jn = jj + LOOKAHEAD

            @pl.when(jn < RPT)
            def _():
                mn = jn // JB
                jjn = jn - mn * JB

                @pl.when(jjn == 0)
                def _():
                    _wait_i()
                    _off_add(mn)

                @pl.when(jnp.logical_and(jjn == 12, mn + 1 < NCH))
                def _():
                    _load_chunk_async(mn + 1)
                _issue_gather(jn)
            return 0
        lax.fori_loop(0, RPT, _edge_step, 0)
        _wait_s()
        plsc.subcore_barrier()

        _zero_ring0()

        def _d_chunk(ch, _):
            nb = nbase0 + ch * DCH
            pltpu.sync_copy(acc_sp.at[pl.ds(nb, DCH)], abf)
            pltpu.sync_copy(ring.at[0, pl.ds(0, DCH)],
                            acc_sp.at[pl.ds(nb, DCH)])
            pltpu.sync_copy(d2_sp.at[pl.ds(nb, DCH)], d2c)

            def _scale_grp(g, _):
                sv = d2c[pl.ds(g * 16, 16)]
                for l in range(16):
                    n = g * 16 + l
                    sc = sv[l]
                    a, b = plsc.unpack(abf[n, pl.ds(0, 32)], format=PK)
                    abf[n, pl.ds(0, 32)] = plsc.pack(a * sc, b * sc, format=PK)
                return 0
            lax.fori_loop(0, DCH // 16, _scale_grp, 0)
            pltpu.sync_copy(abf, t_hbm.at[pl.ds(off + nb, DCH)])
            return 0
        lax.fori_loop(0, NDCH, _d_chunk, 0)
        plsc.subcore_barrier()

        for q in range(BPT // OCH):
            for which in range(2):
                idxv = uidxv if which == 0 else iidxv
                pv_ref = puv if which == 0 else piv
                out_ref = out_u if which == 0 else out_i
                pltpu.sync_copy(t_hbm.at[idxv.at[pl.ds(q * OCH, OCH)]],
                                ring.at[2, pl.ds(0, OCH)])
                pltpu.sync_copy(out_ref.at[c, pl.ds(s * BPT + q * OCH, OCH)],
                                af32)

                def _acc_out(g, _):
                    pv = pv_ref[pl.ds(q * OCH + g * 16, 16)]
                    for l in range(16):
                        r = g * 16 + l
                        pr = pv[l]
                        a, b = plsc.unpack(ring[2, r, pl.ds(0, 32)], format=PK)
                        af32[r, pl.ds(0, 16)] = (af32[r, pl.ds(0, 16)] + pr * a) * fac
                        af32[r, pl.ds(16, 16)] = (af32[r, pl.ds(16, 16)] + pr * b) * fac
                    return 0
                lax.fori_loop(0, OCH // 16, _acc_out, 0)
                pltpu.sync_copy(af32,
                                out_ref.at[c, pl.ds(s * BPT + q * OCH, OCH)])
        return 0
    lax.fori_loop(0, K_LAYERS, _layer, 0)


_sc_call = pl.kernel(
    _sc_body,
    out_type=[
        jax.ShapeDtypeStruct((NC, BATCH, H), jnp.float32),
        jax.ShapeDtypeStruct((NC, BATCH, H), jnp.float32),
        jax.ShapeDtypeStruct((NC * NP, H), BF16),
        jax.ShapeDtypeStruct((NC * NP,), jnp.float32),
    ],
    mesh=plsc.VectorSubcoreMesh(core_axis_name="c", subcore_axis_name="s",
                                num_cores=NC, num_subcores=NS),
    compiler_params=pltpu.CompilerParams(use_tc_tiling_on_sc=False,
                                         needs_layout_passes=False),
    scratch_types=[
        pltpu.VMEM_SHARED((NP, H), BF16),
        pltpu.VMEM_SHARED((NP,), jnp.float32),
        pltpu.VMEM_SHARED((NP,), jnp.float32),
        pltpu.VMEM((RB, B, H), BF16),
        pltpu.VMEM((2, JB, B), jnp.int32),
        pltpu.VMEM((2, CH), jnp.int32),
        pltpu.VMEM((DCH, H), BF16),
        pltpu.VMEM((OCH, H), jnp.float32),
        pltpu.VMEM((DCH,), jnp.float32),
        pltpu.VMEM((DCH,), jnp.float32),
        pltpu.VMEM((DCH,), jnp.float32),
        pltpu.VMEM((NPT,), jnp.float32),
        pltpu.VMEM((B,), jnp.float32),
        pltpu.VMEM((BPT,), jnp.int32),
        pltpu.VMEM((BPT,), jnp.int32),
        pltpu.VMEM((BPT,), jnp.float32),
        pltpu.VMEM((BPT,), jnp.float32),
        pltpu.SemaphoreType.DMA,
        pltpu.SemaphoreType.DMA,
        pltpu.SemaphoreType.DMA,
    ],
)


@jax.jit
def kernel(user_id, item_ids, edge_index, users_emb, items_emb):
    row = edge_index[0].astype(jnp.int32)
    col = edge_index[1].astype(jnp.int32)
    pad = jnp.full((E_PAD - E,), DUMP, dtype=jnp.int32)
    row2d = jnp.concatenate([row, pad]).reshape(E_PAD // B, B)
    col1d = jnp.concatenate([col, pad])

    zpad = jnp.zeros((NP - N_NODES, H), jnp.float32)
    halves = []
    for c in range(NC):
        halves.append(jnp.concatenate([
            users_emb[:, c * H:(c + 1) * H],
            items_emb[:, c * H:(c + 1) * H],
            zpad,
        ], axis=0))
    e0 = jnp.concatenate(halves, axis=0)

    uid = user_id.astype(jnp.int32)
    iid = item_ids.astype(jnp.int32) + NUM_USERS

    out_u, out_i, _t, _p = _sc_call(row2d, col1d, e0, uid, iid)
    return jnp.concatenate([out_u[0], out_u[1], out_i[0], out_i[1]], axis=1)
